# baseline (speedup 1.0000x reference)
import jax
import jax.numpy as jnp
from jax.experimental import pallas as pl
from jax.experimental.pallas import tpu as pltpu

_BI = 1024
_BJ = 1024


def _sim_kernel(y_ref, z_ref, sim_ref, cnt_ref, d_ref, ry_ref):
    i = pl.program_id(0)
    j = pl.program_id(1)
    nj = pl.num_programs(1)
    bi, bj = sim_ref.shape

    yb = y_ref[...]
    zb = z_ref[...]
    zn = zb * (1.0 / jnp.sqrt(jnp.sum(zb * zb, axis=1, keepdims=True)))

    @pl.when(j == 0)
    def _():
        ry_ref[...] = 1.0 / jnp.sqrt(jnp.sum(yb * yb, axis=1, keepdims=True))

    yn = yb * ry_ref[...]
    s = jax.lax.dot_general(
        yn.astype(jnp.bfloat16), zn.astype(jnp.bfloat16),
        (((1,), (1,)), ((), ())), preferred_element_type=jnp.float32)
    sim_ref[...] = s

    @pl.when(j == 0)
    def _():
        row_g = jax.lax.broadcasted_iota(jnp.int32, (bi, bj), 0)
        col_g = jax.lax.broadcasted_iota(jnp.int32, (bi, bj), 1)
        d = jnp.sum(jnp.where(row_g == col_g, s, 0.0), axis=1, keepdims=True)
        d_ref[...] = d
        r = jnp.where(s > d, 1.0, 0.0)
        r = r + jnp.where((s == d) & (col_g < row_g), 1.0, 0.0)
        cnt_ref[...] = jnp.sum(r, axis=1, keepdims=True)

    @pl.when(j != 0)
    def _():
        d = d_ref[...]
        j_actual = jax.lax.rem(i + j, nj)
        tie = jnp.where(j_actual < i, 1.0, 0.0)
        r = jnp.where(s > d, 1.0, 0.0) + jnp.where(s == d, tie, 0.0)
        cnt_ref[...] += jnp.sum(r, axis=1, keepdims=True)


def _acc_kernel(cnt_ref, t1_ref, t10_ref):
    cnt = cnt_ref[...]
    n = cnt.shape[0]
    t1_ref[0, 0] = jnp.sum((cnt == 0.0).astype(jnp.float32)) * (1.0 / n)
    t10_ref[0, 0] = jnp.sum((cnt < 10.0).astype(jnp.float32)) * (1.0 / n)


def kernel(Z, Y):
    b, f = Z.shape
    ni = b // _BI
    nj = b // _BJ

    sim, cnt = pl.pallas_call(
        _sim_kernel,
        grid=(ni, nj),
        in_specs=[
            pl.BlockSpec((_BI, f), lambda i, j: (i, 0)),
            pl.BlockSpec((_BJ, f), lambda i, j: ((i + j) % nj, 0)),
        ],
        out_specs=[
            pl.BlockSpec((_BI, _BJ), lambda i, j: (i, (i + j) % nj)),
            pl.BlockSpec((_BI, 1), lambda i, j: (i, 0)),
        ],
        out_shape=[
            jax.ShapeDtypeStruct((b, b), jnp.float32),
            jax.ShapeDtypeStruct((b, 1), jnp.float32),
        ],
        scratch_shapes=[
            pltpu.VMEM((_BI, 1), jnp.float32),
            pltpu.VMEM((_BI, 1), jnp.float32),
        ],
        compiler_params=pltpu.CompilerParams(
            dimension_semantics=("parallel", "arbitrary"),
            vmem_limit_bytes=62 * 1024 * 1024,
        ),
    )(Y, Z)

    t1, t10 = pl.pallas_call(
        _acc_kernel,
        out_specs=[
            pl.BlockSpec(memory_space=pltpu.SMEM),
            pl.BlockSpec(memory_space=pltpu.SMEM),
        ],
        out_shape=[
            jax.ShapeDtypeStruct((1, 1), jnp.float32),
            jax.ShapeDtypeStruct((1, 1), jnp.float32),
        ],
    )(cnt)

    return (t1[0, 0], t10[0, 0], sim)


# speedup vs baseline: 1.1680x; 1.1680x over previous
import jax
import jax.numpy as jnp
from jax.experimental import pallas as pl
from jax.experimental.pallas import tpu as pltpu

_B = 1024


def _sim_kernel(y_hbm, z_ref, sim_ref, cnt_ref, zn_ref, yn_ref, ystage_ref,
                d_ref, sem):
    i = pl.program_id(0)
    j = pl.program_id(1)
    ni = pl.num_programs(0)
    nj = pl.num_programs(1)
    bi, bj = sim_ref.shape

    def _y_copy(row):
        return pltpu.make_async_copy(
            y_hbm.at[pl.ds(row * bi, bi), :], ystage_ref, sem)

    @pl.when((i == 0) & (j == 0))
    def _():
        _y_copy(0).start()

    @pl.when(j == 0)
    def _():
        _y_copy(i).wait()
        yb = ystage_ref[...]
        yn = yb * (1.0 / jnp.sqrt(jnp.sum(yb * yb, axis=1, keepdims=True)))
        yn_ref[...] = yn.astype(jnp.bfloat16)

    @pl.when((j == 1) & (i + 1 < ni))
    def _():
        _y_copy(i + 1).start()

    @pl.when(i == 0)
    def _():
        zb = z_ref[...]
        zn = zb * (1.0 / jnp.sqrt(jnp.sum(zb * zb, axis=1, keepdims=True)))
        zn_ref[pl.ds(j * bj, bj), :] = zn.astype(jnp.bfloat16)

    j_actual = jax.lax.rem(i + j, nj)
    znb = zn_ref[pl.ds(j_actual * bj, bj), :]
    s = jax.lax.dot_general(
        yn_ref[...], znb, (((1,), (1,)), ((), ())),
        preferred_element_type=jnp.float32)
    sim_ref[...] = s

    @pl.when(j == 0)
    def _():
        row_g = jax.lax.broadcasted_iota(jnp.int32, (bi, bj), 0)
        col_g = jax.lax.broadcasted_iota(jnp.int32, (bi, bj), 1)
        d = jnp.sum(jnp.where(row_g == col_g, s, 0.0), axis=1, keepdims=True)
        d_ref[...] = d
        r = jnp.where(s > d, 1.0, 0.0)
        r = r + jnp.where((s == d) & (col_g < row_g), 1.0, 0.0)
        cnt_ref[...] = jnp.sum(r, axis=1, keepdims=True)

    @pl.when(j != 0)
    def _():
        d = d_ref[...]
        tie = jnp.where(j_actual < i, 1.0, 0.0)
        r = jnp.where(s > d, 1.0, 0.0) + jnp.where(s == d, tie, 0.0)
        cnt_ref[...] += jnp.sum(r, axis=1, keepdims=True)


def _acc_kernel(cnt_ref, t1_ref, t10_ref):
    cnt = cnt_ref[...]
    n = cnt.shape[0]
    t1_ref[0, 0] = jnp.sum((cnt == 0.0).astype(jnp.float32)) * (1.0 / n)
    t10_ref[0, 0] = jnp.sum((cnt < 10.0).astype(jnp.float32)) * (1.0 / n)


def kernel(Z, Y):
    b, f = Z.shape
    ni = b // _B
    nj = b // _B

    sim, cnt = pl.pallas_call(
        _sim_kernel,
        grid=(ni, nj),
        in_specs=[
            pl.BlockSpec(memory_space=pl.ANY),
            pl.BlockSpec((_B, f),
                         lambda i, j: (jnp.where(i == 0, j, nj - 1), 0)),
        ],
        out_specs=[
            pl.BlockSpec((_B, _B), lambda i, j: (i, (i + j) % nj)),
            pl.BlockSpec((_B, 1), lambda i, j: (i, 0)),
        ],
        out_shape=[
            jax.ShapeDtypeStruct((b, b), jnp.float32),
            jax.ShapeDtypeStruct((b, 1), jnp.float32),
        ],
        scratch_shapes=[
            pltpu.VMEM((b, f), jnp.bfloat16),
            pltpu.VMEM((_B, f), jnp.bfloat16),
            pltpu.VMEM((_B, f), jnp.float32),
            pltpu.VMEM((_B, 1), jnp.float32),
            pltpu.SemaphoreType.DMA,
        ],
        compiler_params=pltpu.CompilerParams(
            dimension_semantics=("arbitrary", "arbitrary"),
            vmem_limit_bytes=62 * 1024 * 1024,
        ),
    )(Y, Z)

    t1, t10 = pl.pallas_call(
        _acc_kernel,
        out_specs=[
            pl.BlockSpec(memory_space=pltpu.SMEM),
            pl.BlockSpec(memory_space=pltpu.SMEM),
        ],
        out_shape=[
            jax.ShapeDtypeStruct((1, 1), jnp.float32),
            jax.ShapeDtypeStruct((1, 1), jnp.float32),
        ],
    )(cnt)

    return (t1[0, 0], t10[0, 0], sim)
